# baseline (speedup 1.0000x reference)
import jax
import jax.numpy as jnp
from jax import lax
from jax.experimental import pallas as pl

_HIDDEN = 2048
_EXPERTS = 64
_TM = 1024


def _router_logits_kernel(x_ref, w_ref, o_ref):
    o_ref[...] = lax.dot_general(
        x_ref[...],
        w_ref[...],
        dimension_numbers=(((1,), (1,)), ((), ())),
        preferred_element_type=jnp.float32,
    )


def kernel(hidden_states, weight):
    hs = hidden_states.reshape(-1, _HIDDEN)
    m = hs.shape[0]
    grid = (m // _TM,)
    out = pl.pallas_call(
        _router_logits_kernel,
        grid=grid,
        in_specs=[
            pl.BlockSpec((_TM, _HIDDEN), lambda i: (i, 0)),
            pl.BlockSpec((_EXPERTS, _HIDDEN), lambda i: (0, 0)),
        ],
        out_specs=pl.BlockSpec((_TM, _EXPERTS), lambda i: (i, 0)),
        out_shape=jax.ShapeDtypeStruct((m, _EXPERTS), jnp.float32),
    )(hs, weight)
    return out
